# baseline (speedup 1.0000x reference)
import functools

import jax
import jax.numpy as jnp
from jax import lax
from jax.experimental import pallas as pl
from jax.experimental.pallas import tpu as pltpu
from jax.experimental.pallas import tpu_sc as plsc

B = 4096
L = 200
D = 64
LP = 224
H = LP // 2
CH = H // 16
NC = 2
NS = 16
NW = NC * NS
RPW = B // NW
NV = D // 16
NBUF = 2


def _sc_call(inputs3, emb_table, wtab):
    mesh = plsc.VectorSubcoreMesh(core_axis_name="c", subcore_axis_name="s")

    @functools.partial(
        pl.kernel,
        out_type=jax.ShapeDtypeStruct((B, D), jnp.float32),
        mesh=mesh,
        scratch_types=[
            pltpu.VMEM((RPW, 2, H), jnp.int32),
            pltpu.VMEM((NBUF, 2, H, D), jnp.float32),
            pltpu.VMEM((NBUF, 2 * H), jnp.float32),
            pltpu.VMEM((RPW, D), jnp.float32),
            [pltpu.SemaphoreType.DMA] * NBUF,
        ],
        compiler_params=pltpu.CompilerParams(use_tc_tiling_on_sc=False),
    )
    def k(inputs_hbm, emb_hbm, w_hbm, out_hbm,
          idx_v, rows_v, w_v, res_v, sems):
        wid = lax.axis_index("s") * NC + lax.axis_index("c")
        base = wid * RPW
        pltpu.sync_copy(inputs_hbm.at[pl.ds(base, RPW)], idx_v)

        def issue(row, s):
            for h in range(2):
                pltpu.async_copy(
                    emb_hbm.at[idx_v.at[row, h]], rows_v.at[s, h], sems[s])
                pltpu.async_copy(
                    w_hbm.at[idx_v.at[row, h]],
                    w_v.at[s, pl.ds(h * H, H)], sems[s])

        def drain(s):
            for h in range(2):
                pltpu.make_async_copy(
                    emb_hbm.at[pl.ds(0, H)], rows_v.at[s, h], sems[s]).wait()
                pltpu.make_async_copy(
                    w_hbm.at[pl.ds(0, H)],
                    w_v.at[s, pl.ds(h * H, H)], sems[s]).wait()

        def compute(row, s):
            acc = tuple(jnp.zeros((16,), jnp.float32) for _ in range(NV))

            for h in range(2):
                def c_body_h(c, acc, h=h):
                    l0 = c * 16
                    w16 = w_v[s, pl.ds(h * H + l0, 16)]
                    acc = list(acc)
                    for i in range(16):
                        wi = w16[i]
                        for kv in range(NV):
                            acc[kv] = acc[kv] + wi * rows_v[
                                s, h, l0 + i, pl.ds(kv * 16, 16)]
                    return tuple(acc)
                acc = lax.fori_loop(0, CH, c_body_h, acc)

            for kv in range(NV):
                res_v[row, pl.ds(kv * 16, 16)] = acc[kv]

        issue(0, 0)

        def g_body(g, carry):
            for s in range(NBUF):
                row = g * NBUF + s

                @pl.when(row + 1 < RPW)
                def _():
                    issue(row + 1, (s + 1) % NBUF)

                drain(s)
                compute(row, s)
            return carry

        lax.fori_loop(0, RPW // NBUF, g_body, 0)
        pltpu.sync_copy(res_v, out_hbm.at[pl.ds(base, RPW)])

    return k(inputs3, emb_table, wtab)


def kernel(inputs, emb_table, weight_table):
    wtab = weight_table.at[0, 0].set(0.0).reshape(-1)
    inputs3 = jnp.pad(inputs, ((0, 0), (0, LP - L))).reshape(B, 2, H)
    return _sc_call(inputs3, emb_table, wtab)
